# baseline (speedup 1.0000x reference)
import functools

import jax
import jax.numpy as jnp
from jax import lax
from jax.experimental import pallas as pl
from jax.experimental.pallas import tpu as pltpu
from jax.experimental.pallas import tpu_sc as plsc

N = 10000
D = 128
E = 320000

NC = 2
NS = 16
NW = NC * NS

K = 128
CPT = 80
NBUF = 4
E_PAD = NW * CPT * K
EPT = CPT * K

N_PAD = 10240
ROWS16 = N_PAD // 16
RB = ROWS16 // NS

_mesh = plsc.VectorSubcoreMesh(core_axis_name="c", subcore_axis_name="s")
_sc_params = pltpu.CompilerParams(needs_layout_passes=False)


@functools.partial(
    pl.kernel,
    out_type=jax.ShapeDtypeStruct((NW, N_PAD), jnp.float32),
    mesh=_mesh,
    compiler_params=_sc_params,
    scratch_types=[
        pltpu.VMEM((EPT,), jnp.int32),
        pltpu.VMEM((N_PAD,), jnp.float32),
    ],
)
def _deg_kernel(dst_hbm, deg_out, dstv, hist):
    c = lax.axis_index("c")
    s = lax.axis_index("s")
    w = c * NS + s
    pltpu.sync_copy(dst_hbm.at[w], dstv)

    zero16 = jnp.zeros((16,), jnp.float32)

    def _zero(i, carry):
        hist[pl.ds(i * 16, 16)] = zero16
        return carry

    lax.fori_loop(0, N_PAD // 16, _zero, 0)

    ones16 = jnp.ones((16,), jnp.float32)

    def _count(i, carry):
        idx = dstv[pl.ds(i * 16, 16)]
        plsc.addupdate_scatter(hist, [idx], ones16)
        return carry

    lax.fori_loop(0, EPT // 16, _count, 0)

    pltpu.sync_copy(hist, deg_out.at[w])


@functools.partial(
    pl.kernel,
    out_type=jax.ShapeDtypeStruct((NC, N_PAD, D), jnp.float32),
    mesh=_mesh,
    compiler_params=_sc_params,
    scratch_types=[
        pltpu.VMEM((4, 2, K), jnp.int32),
        pltpu.VMEM((2, K, D), jnp.float32),
        pltpu.VMEM((16, D), jnp.float32),
        pltpu.SemaphoreType.DMA((4,)),
        pltpu.SemaphoreType.DMA((2,)),
        pltpu.SemaphoreType.DMA((2,)),
        pltpu.VMEM_SHARED((N_PAD, D), jnp.float32),
    ],
)
def _agg_kernel(ei_hbm, xs_hbm, out_hbm, idxr, rows, ztile,
                isem, gsem, ssem, aggsh):
    c = lax.axis_index("c")
    s = lax.axis_index("s")
    w = c * NS + s

    zero16 = jnp.zeros((16,), jnp.float32)

    def _zero(t, carry):
        ztile[t // 8, pl.ds((t % 8) * 16, 16)] = zero16
        return carry

    lax.fori_loop(0, 128, _zero, 0)

    rows_per_tile = N_PAD // NS

    def _init(j, carry):
        pltpu.sync_copy(ztile, aggsh.at[pl.ds(s * rows_per_tile + j * 16, 16)])
        return carry

    lax.fori_loop(0, rows_per_tile // 16, _init, 0)
    plsc.subcore_barrier()

    for q in range(4):
        pltpu.async_copy(ei_hbm.at[w, q], idxr.at[q], isem.at[q])

    def _stage(t, q, r):
        pltpu.make_async_copy(
            xs_hbm.at[idxr.at[q, 0]], rows.at[r], gsem.at[r]).wait()
        pltpu.async_copy(
            rows.at[r], aggsh.at[idxr.at[q, 1]], ssem.at[r], add=True)
        pltpu.make_async_copy(
            rows.at[r], aggsh.at[idxr.at[q, 1]], ssem.at[r]).wait()

        q2 = (q + 2) % 4

        @pl.when(t + 2 < CPT)
        def _():
            pltpu.make_async_copy(
                ei_hbm.at[w, t + 2], idxr.at[q2], isem.at[q2]).wait()
            pltpu.async_copy(
                xs_hbm.at[idxr.at[q2, 0]], rows.at[r], gsem.at[r])

        @pl.when(t + 4 < CPT)
        def _():
            pltpu.async_copy(ei_hbm.at[w, t + 4], idxr.at[q], isem.at[q])

    pltpu.make_async_copy(ei_hbm.at[w, 0], idxr.at[0], isem.at[0]).wait()
    pltpu.async_copy(xs_hbm.at[idxr.at[0, 0]], rows.at[0], gsem.at[0])
    pltpu.make_async_copy(ei_hbm.at[w, 1], idxr.at[1], isem.at[1]).wait()
    pltpu.async_copy(xs_hbm.at[idxr.at[1, 0]], rows.at[1], gsem.at[1])

    def _edge(gi, carry):
        t0 = gi * 4
        _stage(t0 + 0, 0, 0)
        _stage(t0 + 1, 1, 1)
        _stage(t0 + 2, 2, 0)
        _stage(t0 + 3, 3, 1)
        return carry

    lax.fori_loop(0, CPT // 4, _edge, 0)
    plsc.subcore_barrier()

    pltpu.sync_copy(
        aggsh.at[pl.ds(s * rows_per_tile, rows_per_tile)],
        out_hbm.at[c, pl.ds(s * rows_per_tile, rows_per_tile)],
    )


def _scale_body(x_ref, h_ref, xs_ref, dinv_ref):
    deg_row = jnp.sum(h_ref[...], axis=0, keepdims=True) + 1.0
    dinv_row = lax.rsqrt(deg_row)
    r = lax.broadcasted_iota(jnp.int32, (128, 128), 0)
    col = lax.broadcasted_iota(jnp.int32, (128, 128), 1)
    diag = jnp.where(r == col, dinv_row, 0.0)
    xs_ref[...] = jnp.dot(diag, x_ref[...],
                          preferred_element_type=jnp.float32)
    dinv_ref[...] = jnp.dot(diag, jnp.ones((128, 1), jnp.float32),
                            preferred_element_type=jnp.float32)


def _scale(x_pad, hists):
    nblk = N_PAD // 128
    return pl.pallas_call(
        _scale_body,
        grid=(nblk,),
        in_specs=[
            pl.BlockSpec((128, D), lambda i: (i, 0)),
            pl.BlockSpec((NW, 128), lambda i: (0, i)),
        ],
        out_specs=[
            pl.BlockSpec((128, D), lambda i: (i, 0)),
            pl.BlockSpec((128, 1), lambda i: (i, 0)),
        ],
        out_shape=[
            jax.ShapeDtypeStruct((N_PAD, D), jnp.float32),
            jax.ShapeDtypeStruct((N_PAD, 1), jnp.float32),
        ],
    )(x_pad, hists)


def _out_body(p0_ref, p1_ref, xs_ref, dinv_ref, w_ref, b_ref, a_ref, o_ref):
    a = (p0_ref[...] + p1_ref[...] + xs_ref[...]) * dinv_ref[...]
    h = jnp.dot(a, w_ref[...], preferred_element_type=jnp.float32)
    h = h + b_ref[...]
    o_ref[...] = jnp.where(h >= 0, h, a_ref[...] * h)


def _finish(p0, p1, xs, dinv, W, b2, a2):
    nblk = N_PAD // 128
    return pl.pallas_call(
        _out_body,
        grid=(nblk,),
        in_specs=[
            pl.BlockSpec((128, D), lambda i: (i, 0)),
            pl.BlockSpec((128, D), lambda i: (i, 0)),
            pl.BlockSpec((128, D), lambda i: (i, 0)),
            pl.BlockSpec((128, 1), lambda i: (i, 0)),
            pl.BlockSpec((D, D), lambda i: (0, 0)),
            pl.BlockSpec((1, D), lambda i: (0, 0)),
            pl.BlockSpec((1, 1), lambda i: (0, 0)),
        ],
        out_specs=pl.BlockSpec((128, D), lambda i: (i, 0)),
        out_shape=jax.ShapeDtypeStruct((N_PAD, D), jnp.float32),
    )(p0, p1, xs, dinv, W, b2, a2)


def kernel(x, edge_index, W, b, prelu_a):
    src = edge_index[0]
    dst = edge_index[1]
    pad = jnp.full((E_PAD - E,), N, dtype=jnp.int32)
    src_p = jnp.concatenate([src, pad]).reshape(NW, CPT, K)
    dst_flat = jnp.concatenate([dst, pad])
    dst_a = dst_flat.reshape(NW, EPT)
    ei = jnp.stack([src_p, dst_flat.reshape(NW, CPT, K)], axis=2)

    x_pad = jnp.pad(x, ((0, N_PAD - N), (0, 0)))

    hists = _deg_kernel(dst_a)
    xs, dinv = _scale(x_pad, hists)

    agg_parts = _agg_kernel(ei, xs)

    out = _finish(agg_parts[0], agg_parts[1], xs, dinv, W,
                  b.reshape(1, D), prelu_a.reshape(1, 1))
    return out[:N]


# speedup vs baseline: 1.3051x; 1.3051x over previous
import functools

import jax
import jax.numpy as jnp
import numpy as np
from jax import lax
from jax.experimental import pallas as pl
from jax.experimental.pallas import tpu as pltpu
from jax.experimental.pallas import tpu_sc as plsc

N = 10000
D = 128
E = 320000

NC = 2
NS = 16
NW = NC * NS

K = 96
CH0 = 112
CH1 = 104
NCH = NS * (CH0 + CH1)
E_PAD = NCH * K
EPT = E_PAD // NW

N_PAD = 10240

_p = np.arange(D)
_g, _r = _p // 32, _p % 32
PERM_SRC = (32 * _g + (_r // 2) + 16 * (_r % 2)).astype(np.int32)
_PMAT = np.zeros((D, D), dtype=np.float32)
_PMAT[PERM_SRC, np.arange(D)] = 1.0

_mesh = plsc.VectorSubcoreMesh(
    core_axis_name="c", subcore_axis_name="s", num_cores=NC, num_subcores=NS)
_sc_params = pltpu.CompilerParams(needs_layout_passes=False)
_sc_params_lin = pltpu.CompilerParams(
    needs_layout_passes=False, use_tc_tiling_on_sc=False)


@functools.partial(
    pl.kernel,
    out_type=jax.ShapeDtypeStruct((NW, N_PAD), jnp.float32),
    mesh=_mesh,
    compiler_params=_sc_params,
    scratch_types=[
        pltpu.VMEM((EPT,), jnp.int32),
        pltpu.VMEM((N_PAD,), jnp.float32),
    ],
)
def _deg_kernel(dst_hbm, deg_out, dstv, hist):
    c = lax.axis_index("c")
    s = lax.axis_index("s")
    w = c * NS + s
    pltpu.sync_copy(dst_hbm.at[w], dstv)

    zero16 = jnp.zeros((16,), jnp.float32)

    def _zero(i, carry):
        hist[pl.ds(i * 16, 16)] = zero16
        return carry

    lax.fori_loop(0, N_PAD // 16, _zero, 0)

    ones16 = jnp.ones((16,), jnp.float32)

    def _count(i, carry):
        idx = dstv[pl.ds(i * 16, 16)]
        plsc.addupdate_scatter(hist, [idx], ones16)
        return carry

    lax.fori_loop(0, EPT // 16, _count, 0)

    pltpu.sync_copy(hist, deg_out.at[w])


@functools.partial(
    pl.kernel,
    out_type=jax.ShapeDtypeStruct((NC, N_PAD, D), jnp.float32),
    mesh=_mesh,
    compiler_params=_sc_params_lin,
    scratch_types=[
        pltpu.VMEM((8, 2, K), jnp.int32),
        pltpu.VMEM((2, K, D // 2), jnp.int32),
        pltpu.VMEM((2, K, D), jnp.float32),
        pltpu.VMEM((16, D), jnp.float32),
        pltpu.SemaphoreType.DMA((8,)),
        pltpu.SemaphoreType.DMA((2,)),
        pltpu.SemaphoreType.DMA((2,)),
        pltpu.VMEM_SHARED((N_PAD, D), jnp.float32),
    ],
)
def _agg_kernel(ei_hbm, xs_hbm, out_hbm, idxr, bbuf, fbuf, ztile,
                isem, gsem, ssem, aggsh):
    c = lax.axis_index("c")
    s = lax.axis_index("s")

    zero16 = jnp.zeros((16,), jnp.float32)

    def _zero(t, carry):
        ztile[t // 8, pl.ds((t % 8) * 16, 16)] = zero16
        return carry

    lax.fori_loop(0, 128, _zero, 0)

    rows_per_tile = N_PAD // NS

    def _init(j, carry):
        pltpu.sync_copy(ztile, aggsh.at[pl.ds(s * rows_per_tile + j * 16, 16)])
        return carry

    lax.fori_loop(0, rows_per_tile // 16, _init, 0)
    plsc.subcore_barrier()

    mask_hi = jnp.full((16,), -65536, jnp.int32)

    def _widen(r):
        def _row(row, carry):
            for g in range(4):
                wv = bbuf.at[r][row, pl.ds(g * 16, 16)]
                lo = plsc.bitcast(wv << 16, jnp.float32)
                hi = plsc.bitcast(lax.bitwise_and(wv, mask_hi), jnp.float32)
                fbuf.at[r][row, pl.ds(g * 32, 16)] = lo
                fbuf.at[r][row, pl.ds(g * 32 + 16, 16)] = hi
            return carry

        lax.fori_loop(0, K, _row, 0)

    def _run(base, cnt):
        def _stage(t, q, r):
            pltpu.make_async_copy(
                xs_hbm.at[idxr.at[q, 0]], bbuf.at[r], gsem.at[r]).wait()

            @pl.when(t >= 2)
            def _():
                pltpu.make_async_copy(
                    fbuf.at[r], aggsh.at[idxr.at[q, 1]], ssem.at[r]).wait()

            _widen(r)

            q2 = (q + 2) % 8

            @pl.when(t + 2 < cnt)
            def _():
                pltpu.make_async_copy(
                    ei_hbm.at[base + t + 2], idxr.at[q2], isem.at[q2]).wait()
                pltpu.async_copy(
                    xs_hbm.at[idxr.at[q2, 0]], bbuf.at[r], gsem.at[r])

            pltpu.async_copy(
                fbuf.at[r], aggsh.at[idxr.at[q, 1]], ssem.at[r], add=True)

            q6 = (q + 6) % 8

            @pl.when(t + 6 < cnt)
            def _():
                pltpu.async_copy(ei_hbm.at[base + t + 6], idxr.at[q6],
                                 isem.at[q6])

        for q in range(6):
            pltpu.async_copy(ei_hbm.at[base + q], idxr.at[q], isem.at[q])
        pltpu.make_async_copy(ei_hbm.at[base], idxr.at[0], isem.at[0]).wait()
        pltpu.async_copy(xs_hbm.at[idxr.at[0, 0]], bbuf.at[0], gsem.at[0])
        pltpu.make_async_copy(ei_hbm.at[base], idxr.at[1], isem.at[1]).wait()
        pltpu.async_copy(xs_hbm.at[idxr.at[1, 0]], bbuf.at[1], gsem.at[1])

        def _edge(gi, carry):
            t0 = gi * 8
            for b in range(8):
                _stage(t0 + b, b, b % 2)
            return carry

        lax.fori_loop(0, cnt // 8, _edge, 0)

        for r in range(2):
            pltpu.make_async_copy(
                fbuf.at[r], aggsh.at[idxr.at[r, 1]], ssem.at[r]).wait()

    @pl.when(c == 0)
    def _():
        _run(s * CH0, CH0)

    @pl.when(c == 1)
    def _():
        _run(NS * CH0 + s * CH1, CH1)

    plsc.subcore_barrier()

    pltpu.sync_copy(
        aggsh.at[pl.ds(s * rows_per_tile, rows_per_tile)],
        out_hbm.at[c, pl.ds(s * rows_per_tile, rows_per_tile)],
    )


_RB_TC = 1024


def _scale_body(x_ref, h_ref, p_ref, xs_ref, dinv_ref):
    deg_row = jnp.sum(h_ref[...], axis=0, keepdims=True) + 1.0
    dinv_row = lax.rsqrt(deg_row)
    r = lax.broadcasted_iota(jnp.int32, (128, 128), 0)
    col = lax.broadcasted_iota(jnp.int32, (128, 128), 1)
    eye = (r == col)
    pmat = p_ref[...]
    for j in range(_RB_TC // 128):
        diag = jnp.where(eye, dinv_row[:, j * 128:(j + 1) * 128], 0.0)
        xsj = jnp.dot(diag, x_ref[j * 128:(j + 1) * 128, :],
                      preferred_element_type=jnp.float32)
        xs_ref[j * 128:(j + 1) * 128, :] = jnp.dot(
            xsj, pmat, preferred_element_type=jnp.float32).astype(jnp.bfloat16)
        dinv_ref[j * 128:(j + 1) * 128, :] = jnp.dot(
            diag, jnp.ones((128, 1), jnp.float32),
            preferred_element_type=jnp.float32)


def _scale(x_pad, hists, pmat):
    nblk = N_PAD // _RB_TC
    return pl.pallas_call(
        _scale_body,
        grid=(nblk,),
        in_specs=[
            pl.BlockSpec((_RB_TC, D), lambda i: (i, 0)),
            pl.BlockSpec((NW, _RB_TC), lambda i: (0, i)),
            pl.BlockSpec((D, D), lambda i: (0, 0)),
        ],
        out_specs=[
            pl.BlockSpec((_RB_TC, D), lambda i: (i, 0)),
            pl.BlockSpec((_RB_TC, 1), lambda i: (i, 0)),
        ],
        out_shape=[
            jax.ShapeDtypeStruct((N_PAD, D), jnp.bfloat16),
            jax.ShapeDtypeStruct((N_PAD, 1), jnp.float32),
        ],
    )(x_pad, hists, pmat)


def _out_body(p0_ref, p1_ref, xs_ref, dinv_ref, w_ref, pt_ref, b_ref, a_ref,
              o_ref):
    xsu = jnp.dot(xs_ref[...].astype(jnp.float32), pt_ref[...],
                  preferred_element_type=jnp.float32)
    a = (p0_ref[...] + p1_ref[...] + xsu) * dinv_ref[...]
    h = jnp.dot(a, w_ref[...], preferred_element_type=jnp.float32)
    h = h + b_ref[...]
    o_ref[...] = jnp.where(h >= 0, h, a_ref[...] * h)


def _finish(p0, p1, xs, dinv, Wp, pmat_t, b2, a2):
    nblk = N_PAD // _RB_TC
    return pl.pallas_call(
        _out_body,
        grid=(nblk,),
        in_specs=[
            pl.BlockSpec((_RB_TC, D), lambda i: (i, 0)),
            pl.BlockSpec((_RB_TC, D), lambda i: (i, 0)),
            pl.BlockSpec((_RB_TC, D), lambda i: (i, 0)),
            pl.BlockSpec((_RB_TC, 1), lambda i: (i, 0)),
            pl.BlockSpec((D, D), lambda i: (0, 0)),
            pl.BlockSpec((D, D), lambda i: (0, 0)),
            pl.BlockSpec((1, D), lambda i: (0, 0)),
            pl.BlockSpec((1, 1), lambda i: (0, 0)),
        ],
        out_specs=pl.BlockSpec((_RB_TC, D), lambda i: (i, 0)),
        out_shape=jax.ShapeDtypeStruct((N, D), jnp.float32),
    )(p0, p1, xs, dinv, Wp, pmat_t, b2, a2)


def kernel(x, edge_index, W, b, prelu_a):
    src = edge_index[0]
    dst = edge_index[1]
    pad = jnp.full((E_PAD - E,), N, dtype=jnp.int32)
    src_flat = jnp.concatenate([src, pad])
    dst_flat = jnp.concatenate([dst, pad])
    dst_a = dst_flat.reshape(NW, EPT)
    ei = jnp.stack([src_flat.reshape(NCH, K), dst_flat.reshape(NCH, K)],
                   axis=1)

    x_pad = jnp.pad(x, ((0, N_PAD - N), (0, 0)))

    hists = _deg_kernel(dst_a)
    xsb, dinv = _scale(x_pad, hists, jnp.asarray(_PMAT))
    xs32 = lax.bitcast_convert_type(
        xsb.reshape(N_PAD, D // 2, 2), jnp.int32)

    agg_parts = _agg_kernel(ei, xs32)

    return _finish(agg_parts[0], agg_parts[1], xsb, dinv, W,
                   jnp.asarray(_PMAT.T), b.reshape(1, D),
                   prelu_a.reshape(1, 1))


# speedup vs baseline: 1.4572x; 1.1165x over previous
import functools

import jax
import jax.numpy as jnp
import numpy as np
from jax import lax
from jax.experimental import pallas as pl
from jax.experimental.pallas import tpu as pltpu
from jax.experimental.pallas import tpu_sc as plsc

N = 10000
D = 128
E = 320000

NC = 2
NS = 16
NW = NC * NS

K = 96
CH0 = 88
CH1 = 128
NCH = NS * (CH0 + CH1)
E_PAD = NCH * K
EPT = E_PAD // NW

N_PAD = 10240


_mesh = plsc.VectorSubcoreMesh(
    core_axis_name="c", subcore_axis_name="s", num_cores=NC, num_subcores=NS)
_sc_params = pltpu.CompilerParams(needs_layout_passes=False)
_sc_params_lin = pltpu.CompilerParams(
    needs_layout_passes=False, use_tc_tiling_on_sc=False)


@functools.partial(
    pl.kernel,
    out_type=jax.ShapeDtypeStruct((NW, N_PAD), jnp.float32),
    mesh=_mesh,
    compiler_params=_sc_params,
    scratch_types=[
        pltpu.VMEM((EPT,), jnp.int32),
        pltpu.VMEM((N_PAD,), jnp.float32),
    ],
)
def _deg_kernel(dst_hbm, deg_out, dstv, hist):
    c = lax.axis_index("c")
    s = lax.axis_index("s")
    w = c * NS + s
    pltpu.sync_copy(dst_hbm.at[w], dstv)

    zero16 = jnp.zeros((16,), jnp.float32)

    def _zero(i, carry):
        hist[pl.ds(i * 16, 16)] = zero16
        return carry

    lax.fori_loop(0, N_PAD // 16, _zero, 0)

    ones16 = jnp.ones((16,), jnp.float32)

    def _count(i, carry):
        idx = dstv[pl.ds(i * 16, 16)]
        plsc.addupdate_scatter(hist, [idx], ones16)
        return carry

    lax.fori_loop(0, EPT // 16, _count, 0)

    pltpu.sync_copy(hist, deg_out.at[w])


@functools.partial(
    pl.kernel,
    out_type=jax.ShapeDtypeStruct((NC, N_PAD, D), jnp.float32),
    mesh=_mesh,
    compiler_params=_sc_params_lin,
    scratch_types=[
        pltpu.VMEM((8, 2, K), jnp.int32),
        pltpu.VMEM((2, K, D // 2), jnp.int32),
        pltpu.VMEM((2, K, D), jnp.float32),
        pltpu.VMEM((16, D), jnp.float32),
        pltpu.SemaphoreType.DMA((8,)),
        pltpu.SemaphoreType.DMA((2,)),
        pltpu.SemaphoreType.DMA((2,)),
        pltpu.VMEM_SHARED((N_PAD, D), jnp.float32),
    ],
)
def _agg_kernel(ei_hbm, xs_hbm, out_hbm, idxr, bbuf, fbuf, ztile,
                isem, gsem, ssem, aggsh):
    c = lax.axis_index("c")
    s = lax.axis_index("s")

    zero16 = jnp.zeros((16,), jnp.float32)

    def _zero(t, carry):
        ztile[t // 8, pl.ds((t % 8) * 16, 16)] = zero16
        return carry

    lax.fori_loop(0, 128, _zero, 0)

    rows_per_tile = N_PAD // NS

    def _init(j, carry):
        pltpu.sync_copy(ztile, aggsh.at[pl.ds(s * rows_per_tile + j * 16, 16)])
        return carry

    lax.fori_loop(0, rows_per_tile // 16, _init, 0)
    plsc.subcore_barrier()

    mask_hi = jnp.full((16,), -65536, jnp.int32)

    def _widen(r):
        def _row(row, carry):
            for g in range(4):
                wv = bbuf.at[r][row, pl.ds(g * 16, 16)]
                lo = plsc.bitcast(wv << 16, jnp.float32)
                hi = plsc.bitcast(lax.bitwise_and(wv, mask_hi), jnp.float32)
                fbuf.at[r][row, pl.ds(g * 32, 16)] = lo
                fbuf.at[r][row, pl.ds(g * 32 + 16, 16)] = hi
            return carry

        lax.fori_loop(0, K, _row, 0)

    def _run(base, cnt):
        def _stage(t, q, r):
            pltpu.make_async_copy(
                xs_hbm.at[idxr.at[q, 0]], bbuf.at[r], gsem.at[r]).wait()

            @pl.when(t >= 2)
            def _():
                pltpu.make_async_copy(
                    fbuf.at[r], aggsh.at[idxr.at[q, 1]], ssem.at[r]).wait()

            _widen(r)

            q2 = (q + 2) % 8

            @pl.when(t + 2 < cnt)
            def _():
                pltpu.make_async_copy(
                    ei_hbm.at[base + t + 2], idxr.at[q2], isem.at[q2]).wait()
                pltpu.async_copy(
                    xs_hbm.at[idxr.at[q2, 0]], bbuf.at[r], gsem.at[r])

            pltpu.async_copy(
                fbuf.at[r], aggsh.at[idxr.at[q, 1]], ssem.at[r], add=True)

            q6 = (q + 6) % 8

            @pl.when(t + 6 < cnt)
            def _():
                pltpu.async_copy(ei_hbm.at[base + t + 6], idxr.at[q6],
                                 isem.at[q6])

        for q in range(6):
            pltpu.async_copy(ei_hbm.at[base + q], idxr.at[q], isem.at[q])
        pltpu.make_async_copy(ei_hbm.at[base], idxr.at[0], isem.at[0]).wait()
        pltpu.async_copy(xs_hbm.at[idxr.at[0, 0]], bbuf.at[0], gsem.at[0])
        pltpu.make_async_copy(ei_hbm.at[base], idxr.at[1], isem.at[1]).wait()
        pltpu.async_copy(xs_hbm.at[idxr.at[1, 0]], bbuf.at[1], gsem.at[1])

        def _edge(gi, carry):
            t0 = gi * 8
            for b in range(8):
                _stage(t0 + b, b, b % 2)
            return carry

        lax.fori_loop(0, cnt // 8, _edge, 0)

        for r in range(2):
            pltpu.make_async_copy(
                fbuf.at[r], aggsh.at[idxr.at[r, 1]], ssem.at[r]).wait()

    @pl.when(c == 0)
    def _():
        _run(s * CH0, CH0)

    @pl.when(c == 1)
    def _():
        _run(NS * CH0 + s * CH1, CH1)

    plsc.subcore_barrier()

    pltpu.sync_copy(
        aggsh.at[pl.ds(s * rows_per_tile, rows_per_tile)],
        out_hbm.at[c, pl.ds(s * rows_per_tile, rows_per_tile)],
    )


_RB_TC = 1024


def _scale_body(x_ref, h_ref, xs_ref, dinv_ref):
    deg_row = jnp.sum(h_ref[...], axis=0, keepdims=True) + 1.0
    dinv_row = lax.rsqrt(deg_row)
    r = lax.broadcasted_iota(jnp.int32, (128, 128), 0)
    col = lax.broadcasted_iota(jnp.int32, (128, 128), 1)
    eye = (r == col)
    r64 = lax.broadcasted_iota(jnp.int32, (128, 64), 0)
    m64 = lax.broadcasted_iota(jnp.int32, (128, 64), 1)
    cols_a = 32 * (m64 // 16) + (m64 % 16)
    sel_a = jnp.where(r64 == cols_a, 1.0, 0.0)
    sel_b = jnp.where(r64 == cols_a + 16, 1.0, 0.0)
    for j in range(_RB_TC // 128):
        diag = jnp.where(eye, dinv_row[:, j * 128:(j + 1) * 128], 0.0)
        xsj = jnp.dot(diag, x_ref[j * 128:(j + 1) * 128, :],
                      preferred_element_type=jnp.float32)
        a_lo = jnp.dot(xsj, sel_a, preferred_element_type=jnp.float32)
        a_hi = jnp.dot(xsj, sel_b, preferred_element_type=jnp.float32)
        lo = lax.bitcast_convert_type(
            a_lo.astype(jnp.bfloat16), jnp.uint16).astype(jnp.int32)
        hi = lax.bitcast_convert_type(
            a_hi.astype(jnp.bfloat16), jnp.uint16).astype(jnp.int32)
        xs_ref[j * 128:(j + 1) * 128, :] = lo | (hi << 16)
        dinv_ref[j * 128:(j + 1) * 128, :] = jnp.dot(
            diag, jnp.ones((128, 1), jnp.float32),
            preferred_element_type=jnp.float32)


def _scale(x_pad, hists):
    nblk = N_PAD // _RB_TC
    return pl.pallas_call(
        _scale_body,
        grid=(nblk,),
        in_specs=[
            pl.BlockSpec((_RB_TC, D), lambda i: (i, 0)),
            pl.BlockSpec((NW, _RB_TC), lambda i: (0, i)),
        ],
        out_specs=[
            pl.BlockSpec((_RB_TC, D // 2), lambda i: (i, 0)),
            pl.BlockSpec((_RB_TC, 1), lambda i: (i, 0)),
        ],
        out_shape=[
            jax.ShapeDtypeStruct((N_PAD, D // 2), jnp.int32),
            jax.ShapeDtypeStruct((N_PAD, 1), jnp.float32),
        ],
    )(x_pad, hists)


def _out_body(p0_ref, p1_ref, xs_ref, dinv_ref, w_ref, b_ref, a_ref, o_ref):
    w32 = xs_ref[...]
    lo = lax.bitcast_convert_type(w32 << 16, jnp.float32)
    hi = lax.bitcast_convert_type(w32 & jnp.int32(-65536), jnp.float32)
    r64 = lax.broadcasted_iota(jnp.int32, (D // 2, D), 0)
    c128 = lax.broadcasted_iota(jnp.int32, (D // 2, D), 1)
    cols_a = 32 * (r64 // 16) + (r64 % 16)
    sel_at = jnp.where(c128 == cols_a, 1.0, 0.0)
    sel_bt = jnp.where(c128 == cols_a + 16, 1.0, 0.0)
    xsu = (jnp.dot(lo, sel_at, preferred_element_type=jnp.float32)
           + jnp.dot(hi, sel_bt, preferred_element_type=jnp.float32))
    a = (p0_ref[...] + p1_ref[...] + xsu) * dinv_ref[...]
    h = jnp.dot(a, w_ref[...], preferred_element_type=jnp.float32)
    h = h + b_ref[...]
    o_ref[...] = jnp.where(h >= 0, h, a_ref[...] * h)


def _finish(p0, p1, xs, dinv, W, b2, a2):
    nblk = N_PAD // _RB_TC
    return pl.pallas_call(
        _out_body,
        grid=(nblk,),
        in_specs=[
            pl.BlockSpec((_RB_TC, D), lambda i: (i, 0)),
            pl.BlockSpec((_RB_TC, D), lambda i: (i, 0)),
            pl.BlockSpec((_RB_TC, D // 2), lambda i: (i, 0)),
            pl.BlockSpec((_RB_TC, 1), lambda i: (i, 0)),
            pl.BlockSpec((D, D), lambda i: (0, 0)),
            pl.BlockSpec((1, D), lambda i: (0, 0)),
            pl.BlockSpec((1, 1), lambda i: (0, 0)),
        ],
        out_specs=pl.BlockSpec((_RB_TC, D), lambda i: (i, 0)),
        out_shape=jax.ShapeDtypeStruct((N, D), jnp.float32),
    )(p0, p1, xs, dinv, W, b2, a2)


def kernel(x, edge_index, W, b, prelu_a):
    src = edge_index[0]
    dst = edge_index[1]
    pad = jnp.full((E_PAD - E,), N, dtype=jnp.int32)
    src_flat = jnp.concatenate([src, pad])
    dst_flat = jnp.concatenate([dst, pad])
    dst_a = dst_flat.reshape(NW, EPT)
    ei = jnp.stack([src_flat.reshape(NCH, K), dst_flat.reshape(NCH, K)],
                   axis=1)

    x_pad = jnp.pad(x, ((0, N_PAD - N), (0, 0)))

    hists = _deg_kernel(dst_a)
    xs32, dinv = _scale(x_pad, hists)

    agg_parts = _agg_kernel(ei, xs32)

    return _finish(agg_parts[0], agg_parts[1], xs32, dinv, W,
                   b.reshape(1, D), prelu_a.reshape(1, 1))


# speedup vs baseline: 1.4848x; 1.0190x over previous
import functools

import jax
import jax.numpy as jnp
import numpy as np
from jax import lax
from jax.experimental import pallas as pl
from jax.experimental.pallas import tpu as pltpu
from jax.experimental.pallas import tpu_sc as plsc

N = 10000
D = 128
E = 320000

NC = 2
NS = 16
NW = NC * NS

K = 96
CH0 = 120
CH1 = 96
NCH = NS * (CH0 + CH1)
E_PAD = NCH * K
EPT = E_PAD // NW

N_PAD = 10240


_mesh = plsc.VectorSubcoreMesh(
    core_axis_name="c", subcore_axis_name="s", num_cores=NC, num_subcores=NS)
_sc_params = pltpu.CompilerParams(needs_layout_passes=False)
_sc_params_lin = pltpu.CompilerParams(
    needs_layout_passes=False, use_tc_tiling_on_sc=False)


@functools.partial(
    pl.kernel,
    out_type=jax.ShapeDtypeStruct((NW, N_PAD), jnp.float32),
    mesh=_mesh,
    compiler_params=_sc_params,
    scratch_types=[
        pltpu.VMEM((EPT,), jnp.int32),
        pltpu.VMEM((N_PAD,), jnp.float32),
    ],
)
def _deg_kernel(dst_hbm, deg_out, dstv, hist):
    c = lax.axis_index("c")
    s = lax.axis_index("s")
    w = c * NS + s
    pltpu.sync_copy(dst_hbm.at[w], dstv)

    zero16 = jnp.zeros((16,), jnp.float32)

    def _zero(i, carry):
        hist[pl.ds(i * 16, 16)] = zero16
        return carry

    lax.fori_loop(0, N_PAD // 16, _zero, 0)

    ones16 = jnp.ones((16,), jnp.float32)

    def _count(i, carry):
        idx = dstv[pl.ds(i * 16, 16)]
        plsc.addupdate_scatter(hist, [idx], ones16)
        return carry

    lax.fori_loop(0, EPT // 16, _count, 0)

    pltpu.sync_copy(hist, deg_out.at[w])


@functools.partial(
    pl.kernel,
    out_type=jax.ShapeDtypeStruct((NC, N_PAD, D), jnp.float32),
    mesh=_mesh,
    compiler_params=_sc_params_lin,
    scratch_types=[
        pltpu.VMEM((8, 2, K), jnp.int32),
        pltpu.VMEM((2, K, D // 2), jnp.int32),
        pltpu.VMEM((2, K, D), jnp.float32),
        pltpu.VMEM((16, D), jnp.float32),
        pltpu.SemaphoreType.DMA((8,)),
        pltpu.SemaphoreType.DMA((2,)),
        pltpu.SemaphoreType.DMA((2,)),
        pltpu.VMEM_SHARED((N_PAD, D), jnp.float32),
    ],
)
def _agg_kernel(ei_hbm, xs_hbm, out_hbm, idxr, bbuf, fbuf, ztile,
                isem, gsem, ssem, aggsh):
    c = lax.axis_index("c")
    s = lax.axis_index("s")

    zero16 = jnp.zeros((16,), jnp.float32)

    def _zero(t, carry):
        ztile[t // 8, pl.ds((t % 8) * 16, 16)] = zero16
        return carry

    lax.fori_loop(0, 128, _zero, 0)

    rows_per_tile = N_PAD // NS

    def _init(j, carry):
        pltpu.sync_copy(ztile, aggsh.at[pl.ds(s * rows_per_tile + j * 16, 16)])
        return carry

    lax.fori_loop(0, rows_per_tile // 16, _init, 0)
    plsc.subcore_barrier()

    mask_hi = jnp.full((16,), -65536, jnp.int32)

    def _widen(r):
        def _row(row, carry):
            for g in range(4):
                wv = bbuf.at[r][row, pl.ds(g * 16, 16)]
                lo = plsc.bitcast(wv << 16, jnp.float32)
                hi = plsc.bitcast(lax.bitwise_and(wv, mask_hi), jnp.float32)
                fbuf.at[r][row, pl.ds(g * 32, 16)] = lo
                fbuf.at[r][row, pl.ds(g * 32 + 16, 16)] = hi
            return carry

        lax.fori_loop(0, K, _row, 0)

    def _run(base, cnt):
        def _stage(t, q, r):
            pltpu.make_async_copy(
                xs_hbm.at[idxr.at[q, 0]], bbuf.at[r], gsem.at[r]).wait()

            @pl.when(t >= 2)
            def _():
                pltpu.make_async_copy(
                    fbuf.at[r], aggsh.at[idxr.at[q, 1]], ssem.at[r]).wait()

            _widen(r)

            q2 = (q + 2) % 8

            @pl.when(t + 2 < cnt)
            def _():
                pltpu.make_async_copy(
                    ei_hbm.at[base + t + 2], idxr.at[q2], isem.at[q2]).wait()
                pltpu.async_copy(
                    xs_hbm.at[idxr.at[q2, 0]], bbuf.at[r], gsem.at[r])

            pltpu.async_copy(
                fbuf.at[r], aggsh.at[idxr.at[q, 1]], ssem.at[r], add=True)

            q6 = (q + 6) % 8

            @pl.when(t + 6 < cnt)
            def _():
                pltpu.async_copy(ei_hbm.at[base + t + 6], idxr.at[q6],
                                 isem.at[q6])

        for q in range(6):
            pltpu.async_copy(ei_hbm.at[base + q], idxr.at[q], isem.at[q])
        pltpu.make_async_copy(ei_hbm.at[base], idxr.at[0], isem.at[0]).wait()
        pltpu.async_copy(xs_hbm.at[idxr.at[0, 0]], bbuf.at[0], gsem.at[0])
        pltpu.make_async_copy(ei_hbm.at[base], idxr.at[1], isem.at[1]).wait()
        pltpu.async_copy(xs_hbm.at[idxr.at[1, 0]], bbuf.at[1], gsem.at[1])

        def _edge(gi, carry):
            t0 = gi * 8
            for b in range(8):
                _stage(t0 + b, b, b % 2)
            return carry

        lax.fori_loop(0, cnt // 8, _edge, 0)

        for r in range(2):
            pltpu.make_async_copy(
                fbuf.at[r], aggsh.at[idxr.at[r, 1]], ssem.at[r]).wait()

    @pl.when(c == 0)
    def _():
        _run(s * CH0, CH0)

    @pl.when(c == 1)
    def _():
        _run(NS * CH0 + s * CH1, CH1)

    plsc.subcore_barrier()

    pltpu.sync_copy(
        aggsh.at[pl.ds(s * rows_per_tile, rows_per_tile)],
        out_hbm.at[c, pl.ds(s * rows_per_tile, rows_per_tile)],
    )


_RB_TC = 1024


def _scale_body(x_ref, h_ref, xs_ref, dinv_ref):
    deg_row = jnp.sum(h_ref[...], axis=0, keepdims=True) + 1.0
    dinv_row = lax.rsqrt(deg_row)
    r = lax.broadcasted_iota(jnp.int32, (128, 128), 0)
    col = lax.broadcasted_iota(jnp.int32, (128, 128), 1)
    eye = (r == col)
    r64 = lax.broadcasted_iota(jnp.int32, (128, 64), 0)
    m64 = lax.broadcasted_iota(jnp.int32, (128, 64), 1)
    cols_a = 32 * (m64 // 16) + (m64 % 16)
    sel_a = jnp.where(r64 == cols_a, 1.0, 0.0)
    sel_b = jnp.where(r64 == cols_a + 16, 1.0, 0.0)
    for j in range(_RB_TC // 128):
        diag = jnp.where(eye, dinv_row[:, j * 128:(j + 1) * 128], 0.0)
        xsj = jnp.dot(diag, x_ref[j * 128:(j + 1) * 128, :],
                      preferred_element_type=jnp.float32)
        a_lo = jnp.dot(xsj, sel_a, preferred_element_type=jnp.float32)
        a_hi = jnp.dot(xsj, sel_b, preferred_element_type=jnp.float32)
        lo = lax.bitcast_convert_type(
            a_lo.astype(jnp.bfloat16), jnp.uint16).astype(jnp.int32)
        hi = lax.bitcast_convert_type(
            a_hi.astype(jnp.bfloat16), jnp.uint16).astype(jnp.int32)
        xs_ref[j * 128:(j + 1) * 128, :] = lo | (hi << 16)
        dinv_ref[j * 128:(j + 1) * 128, :] = jnp.dot(
            diag, jnp.ones((128, 1), jnp.float32),
            preferred_element_type=jnp.float32)


def _scale(x_pad, hists):
    nblk = N_PAD // _RB_TC
    return pl.pallas_call(
        _scale_body,
        grid=(nblk,),
        in_specs=[
            pl.BlockSpec((_RB_TC, D), lambda i: (i, 0)),
            pl.BlockSpec((NW, _RB_TC), lambda i: (0, i)),
        ],
        out_specs=[
            pl.BlockSpec((_RB_TC, D // 2), lambda i: (i, 0)),
            pl.BlockSpec((_RB_TC, 1), lambda i: (i, 0)),
        ],
        out_shape=[
            jax.ShapeDtypeStruct((N_PAD, D // 2), jnp.int32),
            jax.ShapeDtypeStruct((N_PAD, 1), jnp.float32),
        ],
    )(x_pad, hists)


def _out_body(p0_ref, p1_ref, xs_ref, dinv_ref, w_ref, b_ref, a_ref, o_ref):
    w32 = xs_ref[...]
    lo = lax.bitcast_convert_type(w32 << 16, jnp.float32)
    hi = lax.bitcast_convert_type(w32 & jnp.int32(-65536), jnp.float32)
    r64 = lax.broadcasted_iota(jnp.int32, (D // 2, D), 0)
    c128 = lax.broadcasted_iota(jnp.int32, (D // 2, D), 1)
    cols_a = 32 * (r64 // 16) + (r64 % 16)
    sel_at = jnp.where(c128 == cols_a, 1.0, 0.0)
    sel_bt = jnp.where(c128 == cols_a + 16, 1.0, 0.0)
    xsu = (jnp.dot(lo, sel_at, preferred_element_type=jnp.float32)
           + jnp.dot(hi, sel_bt, preferred_element_type=jnp.float32))
    a = (p0_ref[...] + p1_ref[...] + xsu) * dinv_ref[...]
    h = jnp.dot(a, w_ref[...], preferred_element_type=jnp.float32)
    h = h + b_ref[...]
    o_ref[...] = jnp.where(h >= 0, h, a_ref[...] * h)


def _finish(p0, p1, xs, dinv, W, b2, a2):
    nblk = N_PAD // _RB_TC
    return pl.pallas_call(
        _out_body,
        grid=(nblk,),
        in_specs=[
            pl.BlockSpec((_RB_TC, D), lambda i: (i, 0)),
            pl.BlockSpec((_RB_TC, D), lambda i: (i, 0)),
            pl.BlockSpec((_RB_TC, D // 2), lambda i: (i, 0)),
            pl.BlockSpec((_RB_TC, 1), lambda i: (i, 0)),
            pl.BlockSpec((D, D), lambda i: (0, 0)),
            pl.BlockSpec((1, D), lambda i: (0, 0)),
            pl.BlockSpec((1, 1), lambda i: (0, 0)),
        ],
        out_specs=pl.BlockSpec((_RB_TC, D), lambda i: (i, 0)),
        out_shape=jax.ShapeDtypeStruct((N, D), jnp.float32),
    )(p0, p1, xs, dinv, W, b2, a2)


def kernel(x, edge_index, W, b, prelu_a):
    src = edge_index[0]
    dst = edge_index[1]
    pad = jnp.full((E_PAD - E,), N, dtype=jnp.int32)
    src_flat = jnp.concatenate([src, pad])
    dst_flat = jnp.concatenate([dst, pad])
    dst_a = dst_flat.reshape(NW, EPT)
    ei = jnp.stack([src_flat.reshape(NCH, K), dst_flat.reshape(NCH, K)],
                   axis=1)

    x_pad = jnp.pad(x, ((0, N_PAD - N), (0, 0)))

    hists = _deg_kernel(dst_a)
    xs32, dinv = _scale(x_pad, hists)

    agg_parts = _agg_kernel(ei, xs32)

    return _finish(agg_parts[0], agg_parts[1], xs32, dinv, W,
                   b.reshape(1, D), prelu_a.reshape(1, 1))


# speedup vs baseline: 1.4858x; 1.0007x over previous
import functools

import jax
import jax.numpy as jnp
import numpy as np
from jax import lax
from jax.experimental import pallas as pl
from jax.experimental.pallas import tpu as pltpu
from jax.experimental.pallas import tpu_sc as plsc

N = 10000
D = 128
E = 320000

NC = 2
NS = 16
NW = NC * NS

K = 96
CH0 = 128
CH1 = 88
NCH = NS * (CH0 + CH1)
E_PAD = NCH * K
EPT = E_PAD // NW

N_PAD = 10240


_mesh = plsc.VectorSubcoreMesh(
    core_axis_name="c", subcore_axis_name="s", num_cores=NC, num_subcores=NS)
_sc_params = pltpu.CompilerParams(needs_layout_passes=False)
_sc_params_lin = pltpu.CompilerParams(
    needs_layout_passes=False, use_tc_tiling_on_sc=False)


@functools.partial(
    pl.kernel,
    out_type=jax.ShapeDtypeStruct((NW, N_PAD), jnp.float32),
    mesh=_mesh,
    compiler_params=_sc_params,
    scratch_types=[
        pltpu.VMEM((EPT,), jnp.int32),
        pltpu.VMEM((N_PAD,), jnp.float32),
    ],
)
def _deg_kernel(dst_hbm, deg_out, dstv, hist):
    c = lax.axis_index("c")
    s = lax.axis_index("s")
    w = c * NS + s
    pltpu.sync_copy(dst_hbm.at[w], dstv)

    zero16 = jnp.zeros((16,), jnp.float32)

    def _zero(i, carry):
        hist[pl.ds(i * 16, 16)] = zero16
        return carry

    lax.fori_loop(0, N_PAD // 16, _zero, 0)

    ones16 = jnp.ones((16,), jnp.float32)

    def _count(i, carry):
        idx = dstv[pl.ds(i * 16, 16)]
        plsc.addupdate_scatter(hist, [idx], ones16)
        return carry

    lax.fori_loop(0, EPT // 16, _count, 0)

    pltpu.sync_copy(hist, deg_out.at[w])


@functools.partial(
    pl.kernel,
    out_type=jax.ShapeDtypeStruct((NC, N_PAD, D), jnp.float32),
    mesh=_mesh,
    compiler_params=_sc_params_lin,
    scratch_types=[
        pltpu.VMEM((8, 2, K), jnp.int32),
        pltpu.VMEM((2, K, D // 2), jnp.int32),
        pltpu.VMEM((2, K, D), jnp.float32),
        pltpu.VMEM((16, D), jnp.float32),
        pltpu.SemaphoreType.DMA((8,)),
        pltpu.SemaphoreType.DMA((2,)),
        pltpu.SemaphoreType.DMA((2,)),
        pltpu.VMEM_SHARED((N_PAD, D), jnp.float32),
    ],
)
def _agg_kernel(ei_hbm, xs_hbm, out_hbm, idxr, bbuf, fbuf, ztile,
                isem, gsem, ssem, aggsh):
    c = lax.axis_index("c")
    s = lax.axis_index("s")

    zero16 = jnp.zeros((16,), jnp.float32)

    def _zero(t, carry):
        ztile[t // 8, pl.ds((t % 8) * 16, 16)] = zero16
        return carry

    lax.fori_loop(0, 128, _zero, 0)

    rows_per_tile = N_PAD // NS

    def _init(j, carry):
        pltpu.sync_copy(ztile, aggsh.at[pl.ds(s * rows_per_tile + j * 16, 16)])
        return carry

    lax.fori_loop(0, rows_per_tile // 16, _init, 0)
    plsc.subcore_barrier()

    mask_hi = jnp.full((16,), -65536, jnp.int32)

    def _widen(r):
        def _row(row, carry):
            for g in range(4):
                wv = bbuf.at[r][row, pl.ds(g * 16, 16)]
                lo = plsc.bitcast(wv << 16, jnp.float32)
                hi = plsc.bitcast(lax.bitwise_and(wv, mask_hi), jnp.float32)
                fbuf.at[r][row, pl.ds(g * 32, 16)] = lo
                fbuf.at[r][row, pl.ds(g * 32 + 16, 16)] = hi
            return carry

        lax.fori_loop(0, K, _row, 0)

    def _run(base, cnt):
        def _stage(t, q, r):
            pltpu.make_async_copy(
                xs_hbm.at[idxr.at[q, 0]], bbuf.at[r], gsem.at[r]).wait()

            @pl.when(t >= 2)
            def _():
                pltpu.make_async_copy(
                    fbuf.at[r], aggsh.at[idxr.at[q, 1]], ssem.at[r]).wait()

            _widen(r)

            q2 = (q + 2) % 8

            @pl.when(t + 2 < cnt)
            def _():
                pltpu.make_async_copy(
                    ei_hbm.at[base + t + 2], idxr.at[q2], isem.at[q2]).wait()
                pltpu.async_copy(
                    xs_hbm.at[idxr.at[q2, 0]], bbuf.at[r], gsem.at[r])

            pltpu.async_copy(
                fbuf.at[r], aggsh.at[idxr.at[q, 1]], ssem.at[r], add=True)

            q6 = (q + 6) % 8

            @pl.when(t + 6 < cnt)
            def _():
                pltpu.async_copy(ei_hbm.at[base + t + 6], idxr.at[q6],
                                 isem.at[q6])

        for q in range(6):
            pltpu.async_copy(ei_hbm.at[base + q], idxr.at[q], isem.at[q])
        pltpu.make_async_copy(ei_hbm.at[base], idxr.at[0], isem.at[0]).wait()
        pltpu.async_copy(xs_hbm.at[idxr.at[0, 0]], bbuf.at[0], gsem.at[0])
        pltpu.make_async_copy(ei_hbm.at[base], idxr.at[1], isem.at[1]).wait()
        pltpu.async_copy(xs_hbm.at[idxr.at[1, 0]], bbuf.at[1], gsem.at[1])

        def _edge(gi, carry):
            t0 = gi * 8
            for b in range(8):
                _stage(t0 + b, b, b % 2)
            return carry

        lax.fori_loop(0, cnt // 8, _edge, 0)

        for r in range(2):
            pltpu.make_async_copy(
                fbuf.at[r], aggsh.at[idxr.at[r, 1]], ssem.at[r]).wait()

    @pl.when(c == 0)
    def _():
        _run(s * CH0, CH0)

    @pl.when(c == 1)
    def _():
        _run(NS * CH0 + s * CH1, CH1)

    plsc.subcore_barrier()

    pltpu.sync_copy(
        aggsh.at[pl.ds(s * rows_per_tile, rows_per_tile)],
        out_hbm.at[c, pl.ds(s * rows_per_tile, rows_per_tile)],
    )


_RB_TC = 1024


def _scale_body(x_ref, h_ref, xs_ref, dinv_ref):
    deg_row = jnp.sum(h_ref[...], axis=0, keepdims=True) + 1.0
    dinv_row = lax.rsqrt(deg_row)
    r = lax.broadcasted_iota(jnp.int32, (128, 128), 0)
    col = lax.broadcasted_iota(jnp.int32, (128, 128), 1)
    eye = (r == col)
    r64 = lax.broadcasted_iota(jnp.int32, (128, 64), 0)
    m64 = lax.broadcasted_iota(jnp.int32, (128, 64), 1)
    cols_a = 32 * (m64 // 16) + (m64 % 16)
    sel_a = jnp.where(r64 == cols_a, 1.0, 0.0)
    sel_b = jnp.where(r64 == cols_a + 16, 1.0, 0.0)
    for j in range(_RB_TC // 128):
        diag = jnp.where(eye, dinv_row[:, j * 128:(j + 1) * 128], 0.0)
        xsj = jnp.dot(diag, x_ref[j * 128:(j + 1) * 128, :],
                      preferred_element_type=jnp.float32)
        a_lo = jnp.dot(xsj, sel_a, preferred_element_type=jnp.float32)
        a_hi = jnp.dot(xsj, sel_b, preferred_element_type=jnp.float32)
        lo = lax.bitcast_convert_type(
            a_lo.astype(jnp.bfloat16), jnp.uint16).astype(jnp.int32)
        hi = lax.bitcast_convert_type(
            a_hi.astype(jnp.bfloat16), jnp.uint16).astype(jnp.int32)
        xs_ref[j * 128:(j + 1) * 128, :] = lo | (hi << 16)
        dinv_ref[j * 128:(j + 1) * 128, :] = jnp.dot(
            diag, jnp.ones((128, 1), jnp.float32),
            preferred_element_type=jnp.float32)


def _scale(x_pad, hists):
    nblk = N_PAD // _RB_TC
    return pl.pallas_call(
        _scale_body,
        grid=(nblk,),
        in_specs=[
            pl.BlockSpec((_RB_TC, D), lambda i: (i, 0)),
            pl.BlockSpec((NW, _RB_TC), lambda i: (0, i)),
        ],
        out_specs=[
            pl.BlockSpec((_RB_TC, D // 2), lambda i: (i, 0)),
            pl.BlockSpec((_RB_TC, 1), lambda i: (i, 0)),
        ],
        out_shape=[
            jax.ShapeDtypeStruct((N_PAD, D // 2), jnp.int32),
            jax.ShapeDtypeStruct((N_PAD, 1), jnp.float32),
        ],
    )(x_pad, hists)


def _out_body(p0_ref, p1_ref, xs_ref, dinv_ref, w_ref, b_ref, a_ref, o_ref):
    w32 = xs_ref[...]
    lo = lax.bitcast_convert_type(w32 << 16, jnp.float32)
    hi = lax.bitcast_convert_type(w32 & jnp.int32(-65536), jnp.float32)
    r64 = lax.broadcasted_iota(jnp.int32, (D // 2, D), 0)
    c128 = lax.broadcasted_iota(jnp.int32, (D // 2, D), 1)
    cols_a = 32 * (r64 // 16) + (r64 % 16)
    sel_at = jnp.where(c128 == cols_a, 1.0, 0.0)
    sel_bt = jnp.where(c128 == cols_a + 16, 1.0, 0.0)
    xsu = (jnp.dot(lo, sel_at, preferred_element_type=jnp.float32)
           + jnp.dot(hi, sel_bt, preferred_element_type=jnp.float32))
    a = (p0_ref[...] + p1_ref[...] + xsu) * dinv_ref[...]
    h = jnp.dot(a, w_ref[...], preferred_element_type=jnp.float32)
    h = h + b_ref[...]
    o_ref[...] = jnp.where(h >= 0, h, a_ref[...] * h)


def _finish(p0, p1, xs, dinv, W, b2, a2):
    nblk = N_PAD // _RB_TC
    return pl.pallas_call(
        _out_body,
        grid=(nblk,),
        in_specs=[
            pl.BlockSpec((_RB_TC, D), lambda i: (i, 0)),
            pl.BlockSpec((_RB_TC, D), lambda i: (i, 0)),
            pl.BlockSpec((_RB_TC, D // 2), lambda i: (i, 0)),
            pl.BlockSpec((_RB_TC, 1), lambda i: (i, 0)),
            pl.BlockSpec((D, D), lambda i: (0, 0)),
            pl.BlockSpec((1, D), lambda i: (0, 0)),
            pl.BlockSpec((1, 1), lambda i: (0, 0)),
        ],
        out_specs=pl.BlockSpec((_RB_TC, D), lambda i: (i, 0)),
        out_shape=jax.ShapeDtypeStruct((N, D), jnp.float32),
    )(p0, p1, xs, dinv, W, b2, a2)


def kernel(x, edge_index, W, b, prelu_a):
    src = edge_index[0]
    dst = edge_index[1]
    pad = jnp.full((E_PAD - E,), N, dtype=jnp.int32)
    src_flat = jnp.concatenate([src, pad])
    dst_flat = jnp.concatenate([dst, pad])
    dst_a = dst_flat.reshape(NW, EPT)
    ei = jnp.stack([src_flat.reshape(NCH, K), dst_flat.reshape(NCH, K)],
                   axis=1)

    x_pad = jnp.pad(x, ((0, N_PAD - N), (0, 0)))

    hists = _deg_kernel(dst_a)
    xs32, dinv = _scale(x_pad, hists)

    agg_parts = _agg_kernel(ei, xs32)

    return _finish(agg_parts[0], agg_parts[1], xs32, dinv, W,
                   b.reshape(1, D), prelu_a.reshape(1, 1))
